# baseline (speedup 1.0000x reference)
import functools

import jax
import jax.numpy as jnp
from jax import lax
from jax.experimental import pallas as pl
from jax.experimental.pallas import tpu as pltpu
from jax.experimental.pallas import tpu_sc as plsc

N = 10000
NP = 12288
E = 320000
D_IN = 128
DH = 8
DP = 8
ZDP = 16
NC, NS, L = 2, 16, 16
NW = NC * NS
EPW = E // NW
NPW = NP // NW
ECH = 125
NCH = EPW // ECH
RPS = NP // NS

_MESH = plsc.VectorSubcoreMesh(core_axis_name="c", subcore_axis_name="s")
_SC_PARAMS = pltpu.CompilerParams(needs_layout_passes=False,
                                  use_tc_tiling_on_sc=False)


def _wid():
    return lax.axis_index("s") * NC + lax.axis_index("c")


def _rsqrt_newton(x):
    i = lax.bitcast_convert_type(x, jnp.int32)
    y = lax.bitcast_convert_type(jnp.int32(0x5F3759DF) - (i >> 1), jnp.float32)
    for _ in range(3):
        y = y * (1.5 - 0.5 * x * y * y)
    return y


def _mm_body(x_ref, w_ref, o_ref):
    o_ref[...] = jnp.dot(x_ref[...], w_ref[...],
                         preferred_element_type=jnp.float32)


def _matmul(xp, w1p):
    return pl.pallas_call(
        _mm_body,
        grid=(NP // 1024,),
        in_specs=[
            pl.BlockSpec((1024, D_IN), lambda i: (i, 0)),
            pl.BlockSpec((D_IN, ZDP), lambda i: (0, 0)),
        ],
        out_specs=pl.BlockSpec((1024, ZDP), lambda i: (i, 0)),
        out_shape=jax.ShapeDtypeStruct((NP, ZDP), jnp.float32),
    )(xp, w1p)


def _deg_body(srcw, dstw, degs, degd, idxv, h_out, h_in):
    w = _wid()
    zero = jnp.zeros((L,), jnp.float32)

    def zbody(i, _):
        h_out[pl.ds(i * L, L)] = zero
        h_in[pl.ds(i * L, L)] = zero
        return 0

    lax.fori_loop(0, NP // L, zbody, 0, unroll=8)

    one = jnp.full((L,), 1.0, jnp.float32)
    pltpu.sync_copy(srcw.at[w], idxv)

    def sbody(i, _):
        plsc.addupdate_scatter(h_out, [idxv[pl.ds(i * L, L)]], one)
        return 0

    lax.fori_loop(0, EPW // L, sbody, 0, unroll=8)
    pltpu.sync_copy(dstw.at[w], idxv)

    def dbody(i, _):
        plsc.addupdate_scatter(h_in, [idxv[pl.ds(i * L, L)]], one)
        return 0

    lax.fori_loop(0, EPW // L, dbody, 0, unroll=8)
    pltpu.sync_copy(h_out, degs.at[w])
    pltpu.sync_copy(h_in, degd.at[w])


_deg_kernel = functools.partial(
    pl.kernel,
    out_type=(
        jax.ShapeDtypeStruct((NW, NP), jnp.float32),
        jax.ShapeDtypeStruct((NW, NP), jnp.float32),
    ),
    mesh=_MESH,
    compiler_params=_SC_PARAMS,
    scratch_types=[
        pltpu.VMEM((EPW,), jnp.int32),
        pltpu.VMEM((NP,), jnp.float32),
        pltpu.VMEM((NP,), jnp.float32),
    ],
)(_deg_body)


def _norm_body(degs, degd, z1, ns_o, nd_o, y1, psb, pdb, zbuf, ybuf, nsb, ndb):
    w = _wid()
    base = w * NPW
    pltpu.sync_copy(degs.at[:, pl.ds(base, NPW)], psb)
    pltpu.sync_copy(degd.at[:, pl.ds(base, NPW)], pdb)
    pltpu.sync_copy(z1.at[pl.ds(base, NPW)], zbuf)

    def reduce_norm(pb, nb):
        def vbody(i, _):
            acc = jnp.zeros((L,), jnp.float32)
            for k in range(NW):
                acc = acc + pb[k, pl.ds(i * L, L)]
            nrm = _rsqrt_newton(jnp.maximum(acc, 1.0))
            nb[pl.ds(i * L, L)] = jnp.where(acc > 0, nrm, 0.0)
            return 0

        lax.fori_loop(0, NPW // L, vbody, 0)

    reduce_norm(psb, nsb)
    reduce_norm(pdb, ndb)

    ioa = lax.broadcasted_iota(jnp.int32, (L,), 0)

    def ybody(g, _):
        nv = g * L + ioa
        nsv = nsb[pl.ds(g * L, L)]
        for f in range(DH):
            fv = jnp.full((L,), f, jnp.int32)
            v = plsc.load_gather(zbuf, [nv, fv])
            plsc.store_scatter(ybuf, [nv, fv], v * nsv)
        return 0

    lax.fori_loop(0, NPW // L, ybody, 0)
    pltpu.sync_copy(nsb, ns_o.at[pl.ds(base, NPW)])
    pltpu.sync_copy(ndb, nd_o.at[pl.ds(base, NPW)])
    pltpu.sync_copy(ybuf, y1.at[pl.ds(base, NPW)])


_norm_kernel = functools.partial(
    pl.kernel,
    out_type=(
        jax.ShapeDtypeStruct((NP,), jnp.float32),
        jax.ShapeDtypeStruct((NP,), jnp.float32),
        jax.ShapeDtypeStruct((NP, DP), jnp.float32),
    ),
    mesh=_MESH,
    compiler_params=_SC_PARAMS,
    scratch_types=[
        pltpu.VMEM((NW, NPW), jnp.float32),
        pltpu.VMEM((NW, NPW), jnp.float32),
        pltpu.VMEM((NPW, ZDP), jnp.float32),
        pltpu.VMEM((NPW, DP), jnp.float32),
        pltpu.VMEM((NPW,), jnp.float32),
        pltpu.VMEM((NPW,), jnp.float32),
    ],
)(_norm_body)


NBUF = 4


def _p1_body(y1, srcc, dstc, zrs, agg1p, srcb, dstb, msg, aggS, gsem,
             ssem):
    c = lax.axis_index("c")
    s = lax.axis_index("s")
    w = _wid()

    pltpu.sync_copy(zrs, aggS.at[pl.ds(s * RPS, RPS)])
    pltpu.sync_copy(srcc.at[w], srcb)
    pltpu.sync_copy(dstc.at[w], dstb)
    plsc.subcore_barrier()

    def g_start(i, b):
        pltpu.make_async_copy(y1.at[srcb.at[i]], msg.at[b], gsem).start()

    def g_wait(b):
        pltpu.make_async_copy(y1.at[srcb.at[0]], msg.at[b], gsem).wait()

    def s_start(i, b):
        pltpu.make_async_copy(msg.at[b], aggS.at[dstb.at[i]],
                              ssem).start(add=True)

    def s_wait(b):
        pltpu.make_async_copy(msg.at[b], aggS.at[dstb.at[0]], ssem).wait()

    for b in range(NBUF):
        g_start(b, b)

    def ch(gi, _):
        i0 = gi * NBUF
        for b in range(NBUF):
            g_wait(b)
            s_start(i0 + b, b)
        for b in range(NBUF):
            s_wait(b)
            nxt = i0 + NBUF + b

            @pl.when(nxt < NCH)
            def _():
                g_start(nxt, b)

        return 0

    lax.fori_loop(0, NCH // NBUF, ch, 0)
    plsc.subcore_barrier()
    pltpu.sync_copy(aggS.at[pl.ds(s * RPS, RPS)],
                    agg1p.at[c, pl.ds(s * RPS, RPS)])


_p1_kernel = functools.partial(
    pl.kernel,
    out_type=jax.ShapeDtypeStruct((NC, NP, DP), jnp.float32),
    mesh=_MESH,
    compiler_params=_SC_PARAMS,
    scratch_types=[
        pltpu.VMEM((NCH, ECH), jnp.int32),
        pltpu.VMEM((NCH, ECH), jnp.int32),
        pltpu.VMEM((NBUF, ECH, DP), jnp.float32),
        pltpu.VMEM_SHARED((NP, DP), jnp.float32),
        pltpu.SemaphoreType.DMA,
        pltpu.SemaphoreType.DMA,
    ],
)(_p1_body)


def _ew_body(agg1p, ns_i, nd_i, b1p, w2p, y2, p0b, p1b, nsb, ndb, b1b, w2b,
             y2b):
    w = _wid()
    base = w * NPW
    pltpu.sync_copy(agg1p.at[0, pl.ds(base, NPW)], p0b)
    pltpu.sync_copy(agg1p.at[1, pl.ds(base, NPW)], p1b)
    pltpu.sync_copy(ns_i.at[pl.ds(base, NPW)], nsb)
    pltpu.sync_copy(nd_i.at[pl.ds(base, NPW)], ndb)
    pltpu.sync_copy(b1p, b1b)
    pltpu.sync_copy(w2p, w2b)

    b1v = b1b[...]
    w2v = w2b[...]
    ioa = lax.broadcasted_iota(jnp.int32, (L,), 0)

    def nbody(g, _):
        nv = g * L + ioa
        ndv = ndb[pl.ds(g * L, L)]
        nsv = nsb[pl.ds(g * L, L)]
        acc = jnp.zeros((L,), jnp.float32)
        for f in range(DH):
            fv = jnp.full((L,), f, jnp.int32)
            a = plsc.load_gather(p0b, [nv, fv]) + plsc.load_gather(p1b, [nv, fv])
            h = jnp.maximum(a * ndv + b1v[f], 0.0)
            acc = acc + h * w2v[f]
        y2b[pl.ds(g * L, L)] = acc * nsv
        return 0

    lax.fori_loop(0, NPW // L, nbody, 0)
    pltpu.sync_copy(y2b, y2.at[pl.ds(base, NPW)])


_ew_kernel = functools.partial(
    pl.kernel,
    out_type=jax.ShapeDtypeStruct((NP,), jnp.float32),
    mesh=_MESH,
    compiler_params=_SC_PARAMS,
    scratch_types=[
        pltpu.VMEM((NPW, DP), jnp.float32),
        pltpu.VMEM((NPW, DP), jnp.float32),
        pltpu.VMEM((NPW,), jnp.float32),
        pltpu.VMEM((NPW,), jnp.float32),
        pltpu.VMEM((L,), jnp.float32),
        pltpu.VMEM((L,), jnp.float32),
        pltpu.VMEM((NPW,), jnp.float32),
    ],
)(_ew_body)


def _p2_body(y2, srcw, dstw, agg2p, y2b, a2b, sb, db):
    w = _wid()
    pltpu.sync_copy(y2, y2b)
    zero = jnp.zeros((L,), jnp.float32)

    def zbody(i, _):
        a2b[pl.ds(i * L, L)] = zero
        return 0

    lax.fori_loop(0, NP // L, zbody, 0, unroll=8)
    pltpu.sync_copy(srcw.at[w], sb)
    pltpu.sync_copy(dstw.at[w], db)

    def body(i, _):
        v = plsc.load_gather(y2b, [sb[pl.ds(i * L, L)]])
        plsc.addupdate_scatter(a2b, [db[pl.ds(i * L, L)]], v)
        return 0

    lax.fori_loop(0, EPW // L, body, 0, unroll=8)
    pltpu.sync_copy(a2b, agg2p.at[w])


_p2_kernel = functools.partial(
    pl.kernel,
    out_type=jax.ShapeDtypeStruct((NW, NP), jnp.float32),
    mesh=_MESH,
    compiler_params=_SC_PARAMS,
    scratch_types=[
        pltpu.VMEM((NP,), jnp.float32),
        pltpu.VMEM((NP,), jnp.float32),
        pltpu.VMEM((EPW,), jnp.int32),
        pltpu.VMEM((EPW,), jnp.int32),
    ],
)(_p2_body)


def _fin_body(agg2p, nd_i, b2p, outv, pb, ndb, b2b, ob):
    w = _wid()
    base = w * NPW
    pltpu.sync_copy(agg2p.at[:, pl.ds(base, NPW)], pb)
    pltpu.sync_copy(nd_i.at[pl.ds(base, NPW)], ndb)
    pltpu.sync_copy(b2p, b2b)

    def body(i, _):
        acc = jnp.zeros((L,), jnp.float32)
        for k in range(NW):
            acc = acc + pb[k, pl.ds(i * L, L)]
        ob[pl.ds(i * L, L)] = acc * ndb[pl.ds(i * L, L)] + b2b[...]
        return 0

    lax.fori_loop(0, NPW // L, body, 0)
    pltpu.sync_copy(ob, outv.at[pl.ds(base, NPW)])


_fin_kernel = functools.partial(
    pl.kernel,
    out_type=jax.ShapeDtypeStruct((NP,), jnp.float32),
    mesh=_MESH,
    compiler_params=_SC_PARAMS,
    scratch_types=[
        pltpu.VMEM((NW, NPW), jnp.float32),
        pltpu.VMEM((NPW,), jnp.float32),
        pltpu.VMEM((L,), jnp.float32),
        pltpu.VMEM((NPW,), jnp.float32),
    ],
)(_fin_body)


def kernel(inputs, edge_index, W1, b1, W2, b2):
    x = inputs
    ei = edge_index.astype(jnp.int32)
    src, dst = ei[0], ei[1]
    xp = jnp.pad(x, ((0, NP - N), (0, 0)))
    w1p = jnp.pad(W1, ((0, 0), (0, ZDP - DH)))
    b1p = jnp.pad(b1, (0, L - DH))
    w2p = jnp.pad(W2[:, 0], (0, L - DH))
    zrs = jnp.zeros((RPS, DP), jnp.float32)
    b2p = jnp.broadcast_to(b2, (L,))
    srcw = src.reshape(NW, EPW)
    dstw = dst.reshape(NW, EPW)
    srcc = src.reshape(NW, NCH, ECH)
    dstc = dst.reshape(NW, NCH, ECH)

    z1 = _matmul(xp, w1p)
    degs, degd = _deg_kernel(srcw, dstw)
    ns, nd, y1 = _norm_kernel(degs, degd, z1)
    agg1p = _p1_kernel(y1, srcc, dstc, zrs)
    y2 = _ew_kernel(agg1p, ns, nd, b1p, w2p)
    agg2p = _p2_kernel(y2, srcw, dstw)
    outv = _fin_kernel(agg2p, nd, b2p)
    return outv[:N].reshape(N, 1)


# speedup vs baseline: 1.1687x; 1.1687x over previous
import functools

import jax
import jax.numpy as jnp
from jax import lax
from jax.experimental import pallas as pl
from jax.experimental.pallas import tpu as pltpu
from jax.experimental.pallas import tpu_sc as plsc

N = 10000
NP = 12288
E = 320000
D_IN = 128
DH = 8
DP = 8
ZDP = 16
NC, NS, L = 2, 16, 16
NW = NC * NS
EPW = E // NW
NPW = NP // NW
ECH = 125
NCH = EPW // ECH
RPS = NP // NS

_MESH = plsc.VectorSubcoreMesh(core_axis_name="c", subcore_axis_name="s")
_SC_PARAMS = pltpu.CompilerParams(needs_layout_passes=False,
                                  use_tc_tiling_on_sc=False)


def _wid():
    return lax.axis_index("s") * NC + lax.axis_index("c")


def _rsqrt_newton(x):
    i = lax.bitcast_convert_type(x, jnp.int32)
    y = lax.bitcast_convert_type(jnp.int32(0x5F3759DF) - (i >> 1), jnp.float32)
    for _ in range(3):
        y = y * (1.5 - 0.5 * x * y * y)
    return y


def _mm_body(x_ref, w_ref, o_ref):
    o_ref[...] = jnp.dot(x_ref[...], w_ref[...],
                         preferred_element_type=jnp.float32)


def _matmul(xp, w1p):
    return pl.pallas_call(
        _mm_body,
        grid=(NP // 1024,),
        in_specs=[
            pl.BlockSpec((1024, D_IN), lambda i: (i, 0)),
            pl.BlockSpec((D_IN, ZDP), lambda i: (0, 0)),
        ],
        out_specs=pl.BlockSpec((1024, ZDP), lambda i: (i, 0)),
        out_shape=jax.ShapeDtypeStruct((NP, ZDP), jnp.float32),
    )(xp, w1p)


def _deg_body(srcw, dstw, degs, degd, idxv, h_out, h_in):
    w = _wid()
    zero = jnp.zeros((L,), jnp.float32)

    def zbody(i, _):
        h_out[pl.ds(i * L, L)] = zero
        h_in[pl.ds(i * L, L)] = zero
        return 0

    lax.fori_loop(0, NP // L, zbody, 0, unroll=8)

    one = jnp.full((L,), 1.0, jnp.float32)
    pltpu.sync_copy(srcw.at[w], idxv)

    def sbody(i, _):
        plsc.addupdate_scatter(h_out, [idxv[pl.ds(i * L, L)]], one)
        return 0

    lax.fori_loop(0, EPW // L, sbody, 0, unroll=8)
    pltpu.sync_copy(dstw.at[w], idxv)

    def dbody(i, _):
        plsc.addupdate_scatter(h_in, [idxv[pl.ds(i * L, L)]], one)
        return 0

    lax.fori_loop(0, EPW // L, dbody, 0, unroll=8)
    pltpu.sync_copy(h_out, degs.at[w])
    pltpu.sync_copy(h_in, degd.at[w])


_deg_kernel = functools.partial(
    pl.kernel,
    out_type=(
        jax.ShapeDtypeStruct((NW, NP), jnp.float32),
        jax.ShapeDtypeStruct((NW, NP), jnp.float32),
    ),
    mesh=_MESH,
    compiler_params=_SC_PARAMS,
    scratch_types=[
        pltpu.VMEM((EPW,), jnp.int32),
        pltpu.VMEM((NP,), jnp.float32),
        pltpu.VMEM((NP,), jnp.float32),
    ],
)(_deg_body)


def _norm_body(degs, degd, z1, ns_o, nd_o, y1, psb, pdb, zbuf, ybuf, nsb, ndb):
    w = _wid()
    base = w * NPW
    pltpu.sync_copy(degs.at[:, pl.ds(base, NPW)], psb)
    pltpu.sync_copy(degd.at[:, pl.ds(base, NPW)], pdb)
    pltpu.sync_copy(z1.at[pl.ds(base, NPW)], zbuf)

    def reduce_norm(pb, nb):
        def vbody(i, _):
            acc = jnp.zeros((L,), jnp.float32)
            for k in range(NW):
                acc = acc + pb[k, pl.ds(i * L, L)]
            nrm = _rsqrt_newton(jnp.maximum(acc, 1.0))
            nb[pl.ds(i * L, L)] = jnp.where(acc > 0, nrm, 0.0)
            return 0

        lax.fori_loop(0, NPW // L, vbody, 0)

    reduce_norm(psb, nsb)
    reduce_norm(pdb, ndb)

    ioa = lax.broadcasted_iota(jnp.int32, (L,), 0)

    def ybody(g, _):
        nv = g * L + ioa
        nsv = nsb[pl.ds(g * L, L)]
        for f in range(DH):
            fv = jnp.full((L,), f, jnp.int32)
            v = plsc.load_gather(zbuf, [nv, fv])
            plsc.store_scatter(ybuf, [nv, fv], v * nsv)
        return 0

    lax.fori_loop(0, NPW // L, ybody, 0)
    pltpu.sync_copy(nsb, ns_o.at[pl.ds(base, NPW)])
    pltpu.sync_copy(ndb, nd_o.at[pl.ds(base, NPW)])
    pltpu.sync_copy(ybuf, y1.at[pl.ds(base, NPW)])


_norm_kernel = functools.partial(
    pl.kernel,
    out_type=(
        jax.ShapeDtypeStruct((NP,), jnp.float32),
        jax.ShapeDtypeStruct((NP,), jnp.float32),
        jax.ShapeDtypeStruct((NP, DP), jnp.float32),
    ),
    mesh=_MESH,
    compiler_params=_SC_PARAMS,
    scratch_types=[
        pltpu.VMEM((NW, NPW), jnp.float32),
        pltpu.VMEM((NW, NPW), jnp.float32),
        pltpu.VMEM((NPW, ZDP), jnp.float32),
        pltpu.VMEM((NPW, DP), jnp.float32),
        pltpu.VMEM((NPW,), jnp.float32),
        pltpu.VMEM((NPW,), jnp.float32),
    ],
)(_norm_body)


NBUF = 4


def _p1_body(y1, srcc, dstc, zrs, agg1p, srcb, dstb, msg, y1S, aggS, gsem,
             ssem):
    c = lax.axis_index("c")
    s = lax.axis_index("s")
    w = _wid()

    pltpu.sync_copy(zrs, aggS.at[pl.ds(s * RPS, RPS)])
    pltpu.sync_copy(y1.at[pl.ds(s * RPS, RPS)], y1S.at[pl.ds(s * RPS, RPS)])
    pltpu.sync_copy(srcc.at[w], srcb)
    pltpu.sync_copy(dstc.at[w], dstb)
    plsc.subcore_barrier()

    def g_start(i, b):
        pltpu.make_async_copy(y1S.at[srcb.at[i]], msg.at[b], gsem).start()

    def g_wait(b):
        pltpu.make_async_copy(y1S.at[srcb.at[0]], msg.at[b], gsem).wait()

    def s_start(i, b):
        pltpu.make_async_copy(msg.at[b], aggS.at[dstb.at[i]],
                              ssem).start(add=True)

    def s_wait(b):
        pltpu.make_async_copy(msg.at[b], aggS.at[dstb.at[0]], ssem).wait()

    for b in range(NBUF):
        g_start(b, b)

    def ch(gi, _):
        i0 = gi * NBUF
        for b in range(NBUF):
            g_wait(b)
            s_start(i0 + b, b)
        for b in range(NBUF):
            s_wait(b)
            nxt = i0 + NBUF + b

            @pl.when(nxt < NCH)
            def _():
                g_start(nxt, b)

        return 0

    lax.fori_loop(0, NCH // NBUF, ch, 0)
    plsc.subcore_barrier()
    pltpu.sync_copy(aggS.at[pl.ds(s * RPS, RPS)],
                    agg1p.at[c, pl.ds(s * RPS, RPS)])


_p1_kernel = functools.partial(
    pl.kernel,
    out_type=jax.ShapeDtypeStruct((NC, NP, DP), jnp.float32),
    mesh=_MESH,
    compiler_params=_SC_PARAMS,
    scratch_types=[
        pltpu.VMEM((NCH, ECH), jnp.int32),
        pltpu.VMEM((NCH, ECH), jnp.int32),
        pltpu.VMEM((NBUF, ECH, DP), jnp.float32),
        pltpu.VMEM_SHARED((NP, DP), jnp.float32),
        pltpu.VMEM_SHARED((NP, DP), jnp.float32),
        pltpu.SemaphoreType.DMA,
        pltpu.SemaphoreType.DMA,
    ],
)(_p1_body)


def _ew_body(agg1p, ns_i, nd_i, b1p, w2p, y2, p0b, p1b, nsb, ndb, b1b, w2b,
             y2b):
    w = _wid()
    base = w * NPW
    pltpu.sync_copy(agg1p.at[0, pl.ds(base, NPW)], p0b)
    pltpu.sync_copy(agg1p.at[1, pl.ds(base, NPW)], p1b)
    pltpu.sync_copy(ns_i.at[pl.ds(base, NPW)], nsb)
    pltpu.sync_copy(nd_i.at[pl.ds(base, NPW)], ndb)
    pltpu.sync_copy(b1p, b1b)
    pltpu.sync_copy(w2p, w2b)

    b1v = b1b[...]
    w2v = w2b[...]
    ioa = lax.broadcasted_iota(jnp.int32, (L,), 0)

    def nbody(g, _):
        nv = g * L + ioa
        ndv = ndb[pl.ds(g * L, L)]
        nsv = nsb[pl.ds(g * L, L)]
        acc = jnp.zeros((L,), jnp.float32)
        for f in range(DH):
            fv = jnp.full((L,), f, jnp.int32)
            a = plsc.load_gather(p0b, [nv, fv]) + plsc.load_gather(p1b, [nv, fv])
            h = jnp.maximum(a * ndv + b1v[f], 0.0)
            acc = acc + h * w2v[f]
        y2b[pl.ds(g * L, L)] = acc * nsv
        return 0

    lax.fori_loop(0, NPW // L, nbody, 0)
    pltpu.sync_copy(y2b, y2.at[pl.ds(base, NPW)])


_ew_kernel = functools.partial(
    pl.kernel,
    out_type=jax.ShapeDtypeStruct((NP,), jnp.float32),
    mesh=_MESH,
    compiler_params=_SC_PARAMS,
    scratch_types=[
        pltpu.VMEM((NPW, DP), jnp.float32),
        pltpu.VMEM((NPW, DP), jnp.float32),
        pltpu.VMEM((NPW,), jnp.float32),
        pltpu.VMEM((NPW,), jnp.float32),
        pltpu.VMEM((L,), jnp.float32),
        pltpu.VMEM((L,), jnp.float32),
        pltpu.VMEM((NPW,), jnp.float32),
    ],
)(_ew_body)


def _p2_body(y2, srcw, dstw, agg2p, y2b, a2b, sb, db):
    w = _wid()
    pltpu.sync_copy(y2, y2b)
    zero = jnp.zeros((L,), jnp.float32)

    def zbody(i, _):
        a2b[pl.ds(i * L, L)] = zero
        return 0

    lax.fori_loop(0, NP // L, zbody, 0, unroll=8)
    pltpu.sync_copy(srcw.at[w], sb)
    pltpu.sync_copy(dstw.at[w], db)

    def body(i, _):
        v = plsc.load_gather(y2b, [sb[pl.ds(i * L, L)]])
        plsc.addupdate_scatter(a2b, [db[pl.ds(i * L, L)]], v)
        return 0

    lax.fori_loop(0, EPW // L, body, 0, unroll=8)
    pltpu.sync_copy(a2b, agg2p.at[w])


_p2_kernel = functools.partial(
    pl.kernel,
    out_type=jax.ShapeDtypeStruct((NW, NP), jnp.float32),
    mesh=_MESH,
    compiler_params=_SC_PARAMS,
    scratch_types=[
        pltpu.VMEM((NP,), jnp.float32),
        pltpu.VMEM((NP,), jnp.float32),
        pltpu.VMEM((EPW,), jnp.int32),
        pltpu.VMEM((EPW,), jnp.int32),
    ],
)(_p2_body)


def _fin_body(agg2p, nd_i, b2p, outv, pb, ndb, b2b, ob):
    w = _wid()
    base = w * NPW
    pltpu.sync_copy(agg2p.at[:, pl.ds(base, NPW)], pb)
    pltpu.sync_copy(nd_i.at[pl.ds(base, NPW)], ndb)
    pltpu.sync_copy(b2p, b2b)

    def body(i, _):
        acc = jnp.zeros((L,), jnp.float32)
        for k in range(NW):
            acc = acc + pb[k, pl.ds(i * L, L)]
        ob[pl.ds(i * L, L)] = acc * ndb[pl.ds(i * L, L)] + b2b[...]
        return 0

    lax.fori_loop(0, NPW // L, body, 0)
    pltpu.sync_copy(ob, outv.at[pl.ds(base, NPW)])


_fin_kernel = functools.partial(
    pl.kernel,
    out_type=jax.ShapeDtypeStruct((NP,), jnp.float32),
    mesh=_MESH,
    compiler_params=_SC_PARAMS,
    scratch_types=[
        pltpu.VMEM((NW, NPW), jnp.float32),
        pltpu.VMEM((NPW,), jnp.float32),
        pltpu.VMEM((L,), jnp.float32),
        pltpu.VMEM((NPW,), jnp.float32),
    ],
)(_fin_body)


def kernel(inputs, edge_index, W1, b1, W2, b2):
    x = inputs
    ei = edge_index.astype(jnp.int32)
    src, dst = ei[0], ei[1]
    xp = jnp.pad(x, ((0, NP - N), (0, 0)))
    w1p = jnp.pad(W1, ((0, 0), (0, ZDP - DH)))
    b1p = jnp.pad(b1, (0, L - DH))
    w2p = jnp.pad(W2[:, 0], (0, L - DH))
    zrs = jnp.zeros((RPS, DP), jnp.float32)
    b2p = jnp.broadcast_to(b2, (L,))
    srcw = src.reshape(NW, EPW)
    dstw = dst.reshape(NW, EPW)
    srcc = src.reshape(NW, NCH, ECH)
    dstc = dst.reshape(NW, NCH, ECH)

    z1 = _matmul(xp, w1p)
    degs, degd = _deg_kernel(srcw, dstw)
    ns, nd, y1 = _norm_kernel(degs, degd, z1)
    agg1p = _p1_kernel(y1, srcc, dstc, zrs)
    y2 = _ew_kernel(agg1p, ns, nd, b1p, w2p)
    agg2p = _p2_kernel(y2, srcw, dstw)
    outv = _fin_kernel(agg2p, nd, b2p)
    return outv[:N].reshape(N, 1)
